# baseline (speedup 1.0000x reference)
import functools

import jax
import jax.numpy as jnp
from jax.experimental import pallas as pl
from jax.experimental.pallas import tpu as pltpu

TYPE_NUM = 3
N = 4096
D = 128
BR = 512


def _gcn_kernel(g_ref, x_ref, w_ref, b_ref, out_ref, support_ref):
    r = pl.program_id(0)
    t = pl.program_id(1)

    @pl.when(r == 0)
    def _compute_support():
        support_ref[t] = (
            jnp.dot(x_ref[:, t, :], w_ref[t], preferred_element_type=jnp.float32)
            + b_ref[t]
        )

    h = jnp.dot(g_ref[0], support_ref[t], preferred_element_type=jnp.float32)
    contrib = jnp.maximum(h, 0.0) * (1.0 / TYPE_NUM)

    @pl.when(t == 0)
    def _init():
        out_ref[...] = contrib

    @pl.when(t != 0)
    def _acc():
        out_ref[...] += contrib


@jax.jit
def kernel(graphs, features, W, b):
    grid = (N // BR, TYPE_NUM)
    return pl.pallas_call(
        _gcn_kernel,
        grid=grid,
        in_specs=[
            pl.BlockSpec((1, BR, N), lambda r, t: (t, r, 0)),
            pl.BlockSpec((N, TYPE_NUM, D), lambda r, t: (0, 0, 0)),
            pl.BlockSpec((TYPE_NUM, D, D), lambda r, t: (0, 0, 0)),
            pl.BlockSpec((TYPE_NUM, D), lambda r, t: (0, 0)),
        ],
        out_specs=pl.BlockSpec((BR, D), lambda r, t: (r, 0)),
        out_shape=jax.ShapeDtypeStruct((N, D), jnp.float32),
        scratch_shapes=[pltpu.VMEM((TYPE_NUM, N, D), jnp.float32)],
    )(graphs, features, W, b)


# speedup vs baseline: 1.0144x; 1.0144x over previous
import functools

import jax
import jax.numpy as jnp
from jax.experimental import pallas as pl
from jax.experimental.pallas import tpu as pltpu

TYPE_NUM = 3
N = 4096
D = 128
BR = 1024


def _gcn_kernel(g_ref, x_ref, w_ref, b_ref, out_ref, support_ref):
    r = pl.program_id(0)
    t = pl.program_id(1)

    @pl.when(r == 0)
    def _compute_support():
        support_ref[t] = (
            jnp.dot(x_ref[:, t, :], w_ref[t], preferred_element_type=jnp.float32)
            + b_ref[t]
        )

    h = jnp.dot(g_ref[0], support_ref[t], preferred_element_type=jnp.float32)
    contrib = jnp.maximum(h, 0.0) * (1.0 / TYPE_NUM)

    @pl.when(t == 0)
    def _init():
        out_ref[...] = contrib

    @pl.when(t != 0)
    def _acc():
        out_ref[...] += contrib


@jax.jit
def kernel(graphs, features, W, b):
    grid = (N // BR, TYPE_NUM)
    return pl.pallas_call(
        _gcn_kernel,
        grid=grid,
        in_specs=[
            pl.BlockSpec((1, BR, N), lambda r, t: (t, r, 0)),
            pl.BlockSpec((N, TYPE_NUM, D), lambda r, t: (0, 0, 0)),
            pl.BlockSpec((TYPE_NUM, D, D), lambda r, t: (0, 0, 0)),
            pl.BlockSpec((TYPE_NUM, D), lambda r, t: (0, 0)),
        ],
        out_specs=pl.BlockSpec((BR, D), lambda r, t: (r, 0)),
        out_shape=jax.ShapeDtypeStruct((N, D), jnp.float32),
        scratch_shapes=[pltpu.VMEM((TYPE_NUM, N, D), jnp.float32)],
    )(graphs, features, W, b)
